# Initial kernel scaffold; baseline (speedup 1.0000x reference)
import functools

import jax
import jax.numpy as jnp
from jax import lax
from jax.experimental import pallas as pl
from jax.experimental.pallas import tpu as pltpu
from jax.experimental.pallas import tpu_sc as plsc

B = 16384
D = 64
K = 11
NW = 32
BPW = B // NW
CHUNK = 128
NCHUNK = BPW // CHUNK
LANES = 16
GROUPS = CHUNK // LANES


def _sc_scores(center, allidx, in_embed, out_embed):
    mesh = plsc.VectorSubcoreMesh(core_axis_name="c", subcore_axis_name="s")

    @functools.partial(
        pl.kernel,
        mesh=mesh,
        out_type=jax.ShapeDtypeStruct((K, B), jnp.float32),
        scratch_types=[
            pltpu.VMEM((CHUNK,), jnp.int32),
            pltpu.VMEM((K, CHUNK), jnp.int32),
            pltpu.VMEM((CHUNK, D), jnp.float32),
            pltpu.VMEM((K, CHUNK, D), jnp.float32),
            pltpu.VMEM((K, CHUNK), jnp.float32),
            pltpu.SemaphoreType.DMA,
        ],
    )
    def sc_kernel(center_hbm, allidx_hbm, inemb_hbm, outemb_hbm, out_hbm,
                  cidx_v, aidx_v, crow_v, arow_v, sc_v, sem):
        wid = lax.axis_index("s") * 2 + lax.axis_index("c")
        base = wid * BPW

        def chunk_body(ci, carry):
            start = base + ci * CHUNK
            pltpu.sync_copy(center_hbm.at[pl.ds(start, CHUNK)], cidx_v)
            pltpu.sync_copy(allidx_hbm.at[:, pl.ds(start, CHUNK)], aidx_v)
            copies = [pltpu.async_copy(inemb_hbm.at[cidx_v], crow_v, sem)]
            for kk in range(K):
                copies.append(
                    pltpu.async_copy(outemb_hbm.at[aidx_v.at[kk]],
                                     arow_v.at[kk], sem))
            for cp in copies:
                cp.wait()

            def group_body(t, gcarry):
                ridx = t * LANES + lax.iota(jnp.int32, (LANES,))
                acc = [jnp.zeros((LANES,), jnp.float32) for _ in range(K)]
                for d in range(D):
                    didx = jnp.full((LANES,), d, dtype=jnp.int32)
                    cv = plsc.load_gather(crow_v, [ridx, didx])
                    for kk in range(K):
                        kidx = jnp.full((LANES,), kk, dtype=jnp.int32)
                        nv = plsc.load_gather(arow_v, [kidx, ridx, didx])
                        acc[kk] = acc[kk] + cv * nv
                for kk in range(K):
                    sc_v[kk, pl.ds(t * LANES, LANES)] = acc[kk]
                return gcarry

            lax.fori_loop(0, GROUPS, group_body, 0)
            pltpu.sync_copy(sc_v, out_hbm.at[:, pl.ds(start, CHUNK)])
            return carry

        lax.fori_loop(0, NCHUNK, chunk_body, 0)

    return sc_kernel(center, allidx, in_embed, out_embed)


def _tc_loss(scores):
    def body(s_ref, o_ref):
        s = s_ref[...]
        pos = s[0:1, :]
        neg = s[1:K, :]
        total = jnp.sum(jax.nn.log_sigmoid(pos))
        total = total + jnp.sum(jax.nn.log_sigmoid(-neg))
        o_ref[0, 0] = -total / B

    return pl.pallas_call(
        body,
        out_shape=jax.ShapeDtypeStruct((1, 1), jnp.float32),
    )(scores)


def kernel(center, context, negatives, in_embed, out_embed):
    center = center.astype(jnp.int32)
    allidx = jnp.concatenate(
        [context[:, None], negatives], axis=1).astype(jnp.int32).T
    scores = _sc_scores(center, allidx, in_embed, out_embed)
    loss = _tc_loss(scores)
    return loss[0, 0]


# speedup vs baseline: 2.5808x; 2.5808x over previous
import functools

import jax
import jax.numpy as jnp
from jax import lax
from jax.experimental import pallas as pl
from jax.experimental.pallas import tpu as pltpu
from jax.experimental.pallas import tpu_sc as plsc

B = 16384
D = 64
K = 11
NW = 32
BPW = B // NW
CHUNK = 128
NCHUNK = BPW // CHUNK
LANES = 16
GROUPS = CHUNK // LANES


def _sc_scores(center, allidx, in_embed, out_embed):
    mesh = plsc.VectorSubcoreMesh(core_axis_name="c", subcore_axis_name="s")

    @functools.partial(
        pl.kernel,
        mesh=mesh,
        out_type=jax.ShapeDtypeStruct((K, B), jnp.float32),
        scratch_types=[
            pltpu.VMEM((CHUNK,), jnp.int32),
            pltpu.VMEM((K, CHUNK), jnp.int32),
            pltpu.VMEM((CHUNK, D), jnp.float32),
            pltpu.VMEM((K, CHUNK, D), jnp.float32),
            pltpu.VMEM((K, CHUNK), jnp.float32),
            pltpu.SemaphoreType.DMA,
        ],
        compiler_params=pltpu.CompilerParams(
            needs_layout_passes=False, use_tc_tiling_on_sc=False),
    )
    def sc_kernel(center_hbm, allidx_hbm, inemb_hbm, outemb_hbm, out_hbm,
                  cidx_v, aidx_v, crow_v, arow_v, sc_v, sem):
        wid = lax.axis_index("s") * 2 + lax.axis_index("c")
        base = wid * BPW

        def chunk_body(ci, carry):
            start = base + ci * CHUNK
            pltpu.sync_copy(center_hbm.at[pl.ds(start, CHUNK)], cidx_v)
            pltpu.sync_copy(allidx_hbm.at[:, pl.ds(start, CHUNK)], aidx_v)
            copies = [pltpu.async_copy(inemb_hbm.at[cidx_v], crow_v, sem)]
            for kk in range(K):
                copies.append(
                    pltpu.async_copy(outemb_hbm.at[aidx_v.at[kk]],
                                     arow_v.at[kk], sem))
            for cp in copies:
                cp.wait()

            def group_body(t, gcarry):
                ridx = t * LANES + lax.iota(jnp.int32, LANES)
                acc = [jnp.zeros((LANES,), jnp.float32) for _ in range(K)]
                for d in range(D):
                    didx = jnp.full((LANES,), d, dtype=jnp.int32)
                    cv = plsc.load_gather(crow_v, [ridx, didx])
                    for kk in range(K):
                        kidx = jnp.full((LANES,), kk, dtype=jnp.int32)
                        nv = plsc.load_gather(arow_v, [kidx, ridx, didx])
                        acc[kk] = acc[kk] + cv * nv
                for kk in range(K):
                    sc_v[kk, pl.ds(t * LANES, LANES)] = acc[kk]
                return gcarry

            lax.fori_loop(0, GROUPS, group_body, 0)
            pltpu.sync_copy(sc_v, out_hbm.at[:, pl.ds(start, CHUNK)])
            return carry

        lax.fori_loop(0, NCHUNK, chunk_body, 0)

    return sc_kernel(center, allidx, in_embed, out_embed)


def _tc_loss(scores):
    def body(s_ref, o_ref):
        s = s_ref[...]
        pos = s[0:1, :]
        neg = s[1:K, :]
        total = jnp.sum(jax.nn.log_sigmoid(pos))
        total = total + jnp.sum(jax.nn.log_sigmoid(-neg))
        o_ref[...] = jnp.reshape(-total / B, (1, 1))

    return pl.pallas_call(
        body,
        out_shape=jax.ShapeDtypeStruct((1, 1), jnp.float32),
    )(scores)


def kernel(center, context, negatives, in_embed, out_embed):
    center = center.astype(jnp.int32)
    allidx = jnp.concatenate(
        [context[:, None], negatives], axis=1).astype(jnp.int32).T
    scores = _sc_scores(center, allidx, in_embed, out_embed)
    loss = _tc_loss(scores)
    return loss[0, 0]


# speedup vs baseline: 2.5838x; 1.0012x over previous
import functools

import jax
import jax.numpy as jnp
from jax import lax
from jax.experimental import pallas as pl
from jax.experimental.pallas import tpu as pltpu
from jax.experimental.pallas import tpu_sc as plsc

B = 16384
D = 64
NNEG = 10
NW = 32
BPW = B // NW
CHUNK = 128
NCHUNK = BPW // CHUNK
LANES = 16
GROUPS = CHUNK // LANES


def _sc_scores(center, context, negflat, in_embed, out_embed):
    mesh = plsc.VectorSubcoreMesh(core_axis_name="c", subcore_axis_name="s")

    @functools.partial(
        pl.kernel,
        mesh=mesh,
        out_type=(jax.ShapeDtypeStruct((B,), jnp.float32),
                  jax.ShapeDtypeStruct((B * NNEG,), jnp.float32)),
        scratch_types=[
            pltpu.VMEM((CHUNK,), jnp.int32),
            pltpu.VMEM((CHUNK,), jnp.int32),
            pltpu.VMEM((CHUNK * NNEG,), jnp.int32),
            pltpu.VMEM((CHUNK, D), jnp.float32),
            pltpu.VMEM((CHUNK, D), jnp.float32),
            pltpu.VMEM((CHUNK * NNEG, D), jnp.float32),
            pltpu.VMEM((CHUNK,), jnp.float32),
            pltpu.VMEM((CHUNK * NNEG,), jnp.float32),
            pltpu.SemaphoreType.DMA,
        ],
        compiler_params=pltpu.CompilerParams(
            needs_layout_passes=False, use_tc_tiling_on_sc=False),
    )
    def sc_kernel(center_hbm, context_hbm, neg_hbm, inemb_hbm, outemb_hbm,
                  pos_hbm, negsc_hbm,
                  cidx_v, oidx_v, nidx_v, crow_v, orow_v, nrow_v,
                  psc_v, nsc_v, sem):
        wid = lax.axis_index("s") * 2 + lax.axis_index("c")
        base = wid * BPW

        def chunk_body(ci, carry):
            start = base + ci * CHUNK
            pltpu.sync_copy(center_hbm.at[pl.ds(start, CHUNK)], cidx_v)
            pltpu.sync_copy(context_hbm.at[pl.ds(start, CHUNK)], oidx_v)
            pltpu.sync_copy(neg_hbm.at[pl.ds(start * NNEG, CHUNK * NNEG)],
                            nidx_v)
            copies = [
                pltpu.async_copy(inemb_hbm.at[cidx_v], crow_v, sem),
                pltpu.async_copy(outemb_hbm.at[oidx_v], orow_v, sem),
            ]
            for j in range(NNEG):
                copies.append(pltpu.async_copy(
                    outemb_hbm.at[nidx_v.at[pl.ds(j * CHUNK, CHUNK)]],
                    nrow_v.at[pl.ds(j * CHUNK, CHUNK)], sem))
            for cp in copies:
                cp.wait()

            def group_body(t, gcarry):
                ridx = t * LANES + lax.iota(jnp.int32, LANES)
                accp = jnp.zeros((LANES,), jnp.float32)
                accn = [jnp.zeros((LANES,), jnp.float32) for _ in range(NNEG)]
                for d in range(D):
                    didx = jnp.full((LANES,), d, dtype=jnp.int32)
                    cv = plsc.load_gather(crow_v, [ridx, didx])
                    ov = plsc.load_gather(orow_v, [ridx, didx])
                    accp = accp + cv * ov
                    for k in range(NNEG):
                        nv = plsc.load_gather(
                            nrow_v, [ridx * NNEG + k, didx])
                        accn[k] = accn[k] + cv * nv
                psc_v[pl.ds(t * LANES, LANES)] = accp
                for k in range(NNEG):
                    plsc.store_scatter(nsc_v, [ridx * NNEG + k], accn[k])
                return gcarry

            lax.fori_loop(0, GROUPS, group_body, 0)
            pltpu.sync_copy(psc_v, pos_hbm.at[pl.ds(start, CHUNK)])
            pltpu.sync_copy(nsc_v,
                            negsc_hbm.at[pl.ds(start * NNEG, CHUNK * NNEG)])
            return carry

        lax.fori_loop(0, NCHUNK, chunk_body, 0)

    return sc_kernel(center, context, negflat, in_embed, out_embed)


def _tc_loss(pos, neg):
    def body(p_ref, n_ref, o_ref):
        total = jnp.sum(jax.nn.log_sigmoid(p_ref[...]))
        total = total + jnp.sum(jax.nn.log_sigmoid(-n_ref[...]))
        o_ref[...] = jnp.reshape(-total / B, (1, 1))

    return pl.pallas_call(
        body,
        out_shape=jax.ShapeDtypeStruct((1, 1), jnp.float32),
    )(pos, neg)


def kernel(center, context, negatives, in_embed, out_embed):
    center = center.astype(jnp.int32)
    context = context.astype(jnp.int32)
    negflat = negatives.astype(jnp.int32).reshape(B * NNEG)
    pos, neg = _sc_scores(center, context, negflat, in_embed, out_embed)
    loss = _tc_loss(pos.reshape(128, B // 128),
                    neg.reshape(1280, B // 128))
    return loss[0, 0]


# speedup vs baseline: 2.6732x; 1.0346x over previous
import functools

import jax
import jax.numpy as jnp
from jax import lax
from jax.experimental import pallas as pl
from jax.experimental.pallas import tpu as pltpu
from jax.experimental.pallas import tpu_sc as plsc

B = 16384
D = 64
DP = 128
NNEG = 10
NW = 32
BPW = B // NW
CHUNK = 64
NCHUNK = BPW // CHUNK
LANES = 16
GROUPS = CHUNK // LANES


def _sc_scores(center, context, negflat, inp, outp):
    mesh = plsc.VectorSubcoreMesh(core_axis_name="c", subcore_axis_name="s")

    @functools.partial(
        pl.kernel,
        mesh=mesh,
        out_type=(jax.ShapeDtypeStruct((B,), jnp.float32),
                  jax.ShapeDtypeStruct((B * NNEG,), jnp.float32)),
        scratch_types=[
            pltpu.VMEM((CHUNK,), jnp.int32),
            pltpu.VMEM((CHUNK,), jnp.int32),
            pltpu.VMEM((CHUNK * NNEG,), jnp.int32),
            pltpu.VMEM((CHUNK, DP), jnp.float32),
            pltpu.VMEM((CHUNK, DP), jnp.float32),
            pltpu.VMEM((CHUNK * NNEG, DP), jnp.float32),
            pltpu.VMEM((CHUNK,), jnp.float32),
            pltpu.VMEM((CHUNK * NNEG,), jnp.float32),
            pltpu.SemaphoreType.DMA,
        ],
        compiler_params=pltpu.CompilerParams(
            needs_layout_passes=False, use_tc_tiling_on_sc=True),
    )
    def sc_kernel(center_hbm, context_hbm, neg_hbm, inemb_hbm, outemb_hbm,
                  pos_hbm, negsc_hbm,
                  cidx_v, oidx_v, nidx_v, crow_v, orow_v, nrow_v,
                  psc_v, nsc_v, sem):
        wid = lax.axis_index("s") * 2 + lax.axis_index("c")
        base = wid * BPW

        def chunk_body(ci, carry):
            start = base + ci * CHUNK
            pltpu.sync_copy(center_hbm.at[pl.ds(start, CHUNK)], cidx_v)
            pltpu.sync_copy(context_hbm.at[pl.ds(start, CHUNK)], oidx_v)
            pltpu.sync_copy(neg_hbm.at[pl.ds(start * NNEG, CHUNK * NNEG)],
                            nidx_v)
            copies = [
                pltpu.async_copy(inemb_hbm.at[cidx_v], crow_v, sem),
                pltpu.async_copy(outemb_hbm.at[oidx_v], orow_v, sem),
            ]
            for j in range(NNEG):
                copies.append(pltpu.async_copy(
                    outemb_hbm.at[nidx_v.at[pl.ds(j * CHUNK, CHUNK)]],
                    nrow_v.at[pl.ds(j * CHUNK, CHUNK)], sem))
            for cp in copies:
                cp.wait()

            def group_body(t, gcarry):
                ridx = t * LANES + lax.iota(jnp.int32, LANES)
                accp = jnp.zeros((LANES,), jnp.float32)
                accn = [jnp.zeros((LANES,), jnp.float32) for _ in range(NNEG)]
                for d in range(D):
                    didx = jnp.full((LANES,), d, dtype=jnp.int32)
                    cv = plsc.load_gather(crow_v, [ridx, didx])
                    ov = plsc.load_gather(orow_v, [ridx, didx])
                    accp = accp + cv * ov
                    for k in range(NNEG):
                        nv = plsc.load_gather(
                            nrow_v, [ridx * NNEG + k, didx])
                        accn[k] = accn[k] + cv * nv
                psc_v[pl.ds(t * LANES, LANES)] = accp
                for k in range(NNEG):
                    plsc.store_scatter(nsc_v, [ridx * NNEG + k], accn[k])
                return gcarry

            lax.fori_loop(0, GROUPS, group_body, 0)
            pltpu.sync_copy(psc_v, pos_hbm.at[pl.ds(start, CHUNK)])
            pltpu.sync_copy(nsc_v,
                            negsc_hbm.at[pl.ds(start * NNEG, CHUNK * NNEG)])
            return carry

        lax.fori_loop(0, NCHUNK, chunk_body, 0)

    return sc_kernel(center, context, negflat, inp, outp)


def _tc_loss(pos, neg):
    def body(p_ref, n_ref, o_ref):
        total = jnp.sum(jax.nn.log_sigmoid(p_ref[...]))
        total = total + jnp.sum(jax.nn.log_sigmoid(-n_ref[...]))
        o_ref[...] = jnp.reshape(-total / B, (1, 1))

    return pl.pallas_call(
        body,
        out_shape=jax.ShapeDtypeStruct((1, 1), jnp.float32),
    )(pos, neg)


def kernel(center, context, negatives, in_embed, out_embed):
    center = center.astype(jnp.int32)
    context = context.astype(jnp.int32)
    negflat = negatives.astype(jnp.int32).reshape(B * NNEG)
    inp = jnp.pad(in_embed, ((0, 0), (0, DP - D)))
    outp = jnp.pad(out_embed, ((0, 0), (0, DP - D)))
    pos, neg = _sc_scores(center, context, negflat, inp, outp)
    loss = _tc_loss(pos.reshape(128, B // 128),
                    neg.reshape(1280, B // 128))
    return loss[0, 0]
